# baseline (speedup 1.0000x reference)
import jax
import jax.numpy as jnp
from jax.experimental import pallas as pl
from jax.experimental.pallas import tpu as pltpu

_ALPHA = 1.0
_TOPK = 3
_BLOCK = 1024
_NEG = -1e9


def _hyper_kernel(mask_ref, ne_ref, hy_ref, val_ref, idx_ref):
    i = pl.program_id(0)
    b = ne_ref.shape[0]
    ne = ne_ref[...]
    hy = hy_ref[...]
    simt = jax.lax.dot_general(
        hy, ne, (((1,), (1,)), ((), ())),
        preferred_element_type=jnp.float32)
    simt = jnp.maximum(_ALPHA * simt, 0.0)
    avg = jnp.mean(mask_ref[...], axis=0)
    simt = jnp.where(avg[None, :] < 0.5, _NEG, simt)
    m = jnp.max(simt, axis=0, keepdims=True)
    e = jnp.exp(simt - m)
    soft = e / jnp.sum(e, axis=0, keepdims=True)

    h = soft.shape[0]
    row = jax.lax.broadcasted_iota(jnp.int32, soft.shape, 0)
    v = soft
    for k in range(_TOPK):
        mk = jnp.max(v, axis=0)
        ik = jnp.min(jnp.where(v == mk[None, :], row, h), axis=0)
        val_ref[k, :] = mk
        idx_ref[k, :] = ik
        v = jnp.where(row == ik[None, :], -1.0, v)


def kernel(features, mask, node_embeds, hyper_embeds):
    seq_len = min(features.shape[1], node_embeds.shape[0])
    ne = node_embeds[:seq_len]
    dim = ne.shape[1]
    hnum = hyper_embeds.shape[0]
    nblk = seq_len // _BLOCK

    vals, idxs = pl.pallas_call(
        _hyper_kernel,
        grid=(nblk,),
        in_specs=[
            pl.BlockSpec((mask.shape[0], _BLOCK), lambda i: (0, i)),
            pl.BlockSpec((_BLOCK, dim), lambda i: (i, 0)),
            pl.BlockSpec((hnum, dim), lambda i: (0, 0)),
        ],
        out_specs=[
            pl.BlockSpec((_TOPK, _BLOCK), lambda i: (0, i)),
            pl.BlockSpec((_TOPK, _BLOCK), lambda i: (0, i)),
        ],
        out_shape=[
            jax.ShapeDtypeStruct((_TOPK, seq_len), jnp.float32),
            jax.ShapeDtypeStruct((_TOPK, seq_len), jnp.int32),
        ],
    )(mask, ne, hyper_embeds)

    n_edges = _TOPK * seq_len
    l = jax.lax.iota(jnp.int32, n_edges)
    rows = l // _TOPK
    src = (l % _TOPK) * seq_len + rows
    edge_weight = jnp.take(vals.reshape(-1), src)
    cols = jnp.take(idxs.reshape(-1), src)
    edge_index = jnp.stack([rows, cols], axis=0)
    return (edge_index, edge_weight)


# speedup vs baseline: 1.3783x; 1.3783x over previous
import functools

import jax
import jax.numpy as jnp
from jax import lax
from jax.experimental import pallas as pl
from jax.experimental.pallas import tpu as pltpu
from jax.experimental.pallas import tpu_sc as plsc

_ALPHA = 1.0
_TOPK = 3
_BLOCK = 1024
_NEG = -1e9


def _hyper_kernel(mask_ref, ne_ref, hy_ref, val_ref, idx_ref):
    ne = ne_ref[...]
    hy = hy_ref[...]
    simt = jax.lax.dot_general(
        hy, ne, (((1,), (1,)), ((), ())),
        preferred_element_type=jnp.float32)
    simt = jnp.maximum(_ALPHA * simt, 0.0)
    avg = jnp.mean(mask_ref[...], axis=0)
    simt = jnp.where(avg[None, :] < 0.5, _NEG, simt)
    m = jnp.max(simt, axis=0, keepdims=True)
    e = jnp.exp(simt - m)
    soft = e / jnp.sum(e, axis=0, keepdims=True)

    h = soft.shape[0]
    row = jax.lax.broadcasted_iota(jnp.int32, soft.shape, 0)
    v = soft
    for k in range(_TOPK):
        mk = jnp.max(v, axis=0)
        ik = jnp.min(jnp.where(v == mk[None, :], row, h), axis=0)
        val_ref[k, :] = mk
        idx_ref[k, :] = ik
        v = jnp.where(row == ik[None, :], -1.0, v)


_NTILES = 32


def _edge_assemble_sc(vals, idxs, seq_len):
    x, xi = vals, idxs
    jpw = seq_len // _NTILES
    epw = 3 * jpw
    n_edges = 3 * seq_len
    mesh = plsc.VectorSubcoreMesh(core_axis_name="c", subcore_axis_name="s")

    @functools.partial(
        pl.kernel,
        out_type=[jax.ShapeDtypeStruct((n_edges,), jnp.float32),
                  jax.ShapeDtypeStruct((2, n_edges), jnp.int32)],
        mesh=mesh,
        compiler_params=pltpu.CompilerParams(needs_layout_passes=False),
        scratch_types=[pltpu.VMEM((3, jpw + 16), jnp.float32),
                       pltpu.VMEM((3, jpw + 16), jnp.int32),
                       pltpu.VMEM((epw,), jnp.float32),
                       pltpu.VMEM((1, epw), jnp.int32),
                       pltpu.VMEM((1, epw), jnp.int32)],
    )
    def _sc(x_hbm, xi_hbm, ew_hbm, ei_hbm, vv, vi, ewv, e0, e1):
        w = lax.axis_index("s") * 2 + lax.axis_index("c")
        jbase = w * jpw
        obase = w * epw
        for k in range(3):
            pltpu.sync_copy(x_hbm.at[pl.ds(k, 1), pl.ds(jbase, jpw)],
                            vv.at[pl.ds(k, 1), pl.ds(0, jpw)])
            pltpu.sync_copy(xi_hbm.at[pl.ds(k, 1), pl.ds(jbase, jpw)],
                            vi.at[pl.ds(k, 1), pl.ds(0, jpw)])
        iota = lax.iota(jnp.int32, 16)
        for g in range(jpw // 16):
            permf, permi = [], []
            for k in range(3):
                keys = (3 * iota + k) & 15
                _, pf = plsc.sort_key_val(keys, vv[k, pl.ds(g * 16, 16)])
                _, pi = plsc.sort_key_val(keys, vi[k, pl.ds(g * 16, 16)])
                permf.append(pf)
                permi.append(pi)
            for m in range(3):
                r = 16 * m + iota
                q = (r * 21846) >> 16
                kk = r - 3 * q
                outf = jnp.where(kk == 0, permf[0],
                                 jnp.where(kk == 1, permf[1], permf[2]))
                outi = jnp.where(kk == 0, permi[0],
                                 jnp.where(kk == 1, permi[1], permi[2]))
                o = 48 * g + 16 * m
                node = jbase + 16 * g + q
                ewv[pl.ds(o, 16)] = outf
                e0[0, pl.ds(o, 16)] = node
                e1[0, pl.ds(o, 16)] = outi
        pltpu.sync_copy(ewv, ew_hbm.at[pl.ds(obase, epw)])
        pltpu.sync_copy(e0, ei_hbm.at[pl.ds(0, 1), pl.ds(obase, epw)])
        pltpu.sync_copy(e1, ei_hbm.at[pl.ds(1, 1), pl.ds(obase, epw)])

    ew, ei = _sc(x, xi)
    return ei, ew


def kernel(features, mask, node_embeds, hyper_embeds):
    seq_len = min(features.shape[1], node_embeds.shape[0])
    ne = node_embeds[:seq_len]
    dim = ne.shape[1]
    hnum = hyper_embeds.shape[0]
    nblk = seq_len // _BLOCK

    vals, idxs = pl.pallas_call(
        _hyper_kernel,
        grid=(nblk,),
        in_specs=[
            pl.BlockSpec((mask.shape[0], _BLOCK), lambda i: (0, i)),
            pl.BlockSpec((_BLOCK, dim), lambda i: (i, 0)),
            pl.BlockSpec((hnum, dim), lambda i: (0, 0)),
        ],
        out_specs=[
            pl.BlockSpec((_TOPK, _BLOCK), lambda i: (0, i)),
            pl.BlockSpec((_TOPK, _BLOCK), lambda i: (0, i)),
        ],
        out_shape=[
            jax.ShapeDtypeStruct((_TOPK, seq_len), jnp.float32),
            jax.ShapeDtypeStruct((_TOPK, seq_len), jnp.int32),
        ],
    )(mask, ne, hyper_embeds)

    edge_index, edge_weight = _edge_assemble_sc(vals, idxs, seq_len)
    return (edge_index, edge_weight)


# speedup vs baseline: 1.4947x; 1.0845x over previous
import functools

import jax
import jax.numpy as jnp
from jax import lax
from jax.experimental import pallas as pl
from jax.experimental.pallas import tpu as pltpu
from jax.experimental.pallas import tpu_sc as plsc

_ALPHA = 1.0
_TOPK = 3
_BLOCK = 1024
_NEG = -1e9


def _hyper_kernel(mask_ref, ne_ref, hy_ref, val_ref, idx_ref):
    ne = ne_ref[...]
    hy = hy_ref[...]
    simt = jax.lax.dot_general(
        hy, ne, (((1,), (1,)), ((), ())),
        preferred_element_type=jnp.float32)
    simt = jnp.maximum(_ALPHA * simt, 0.0)
    avg = jnp.mean(mask_ref[...], axis=0)
    simt = jnp.where(avg[None, :] < 0.5, _NEG, simt)
    m = jnp.max(simt, axis=0, keepdims=True)
    e = jnp.exp(simt - m)
    soft = e / jnp.sum(e, axis=0, keepdims=True)

    h = soft.shape[0]
    row = jax.lax.broadcasted_iota(jnp.int32, soft.shape, 0)
    v = soft
    for k in range(_TOPK):
        mk = jnp.max(v, axis=0)
        ik = jnp.min(jnp.where(v == mk[None, :], row, h), axis=0)
        val_ref[k, :] = mk
        idx_ref[k, :] = ik
        v = jnp.where(row == ik[None, :], -1.0, v)


_NTILES = 32


def _edge_assemble_sc(vals, idxs, seq_len):
    x, xi = vals, idxs
    jpw = seq_len // _NTILES
    epw = 3 * jpw
    n_edges = 3 * seq_len
    mesh = plsc.VectorSubcoreMesh(core_axis_name="c", subcore_axis_name="s")

    @functools.partial(
        pl.kernel,
        out_type=[jax.ShapeDtypeStruct((n_edges,), jnp.float32),
                  jax.ShapeDtypeStruct((2, n_edges), jnp.int32)],
        mesh=mesh,
        compiler_params=pltpu.CompilerParams(needs_layout_passes=False),
        scratch_types=[pltpu.VMEM((3, jpw), jnp.float32),
                       pltpu.VMEM((3, jpw), jnp.int32),
                       pltpu.VMEM((epw,), jnp.float32),
                       pltpu.VMEM((1, epw), jnp.int32),
                       pltpu.VMEM((1, epw), jnp.int32),
                       pltpu.SemaphoreType.DMA,
                       pltpu.SemaphoreType.DMA],
    )
    def _sc(x_hbm, xi_hbm, ew_hbm, ei_hbm, vv, vi, ewv, e0, e1, s1, s2):
        w = lax.axis_index("s") * 2 + lax.axis_index("c")
        jbase = w * jpw
        obase = w * epw
        c1 = pltpu.async_copy(x_hbm.at[pl.ds(0, 3), pl.ds(jbase, jpw)], vv, s1)
        c2 = pltpu.async_copy(xi_hbm.at[pl.ds(0, 3), pl.ds(jbase, jpw)], vi, s2)
        c1.wait()
        c2.wait()
        iota = lax.iota(jnp.int32, 16)
        for g in range(jpw // 16):
            permf, permi = [], []
            for k in range(3):
                keys = (3 * iota + k) & 15
                _, pf = plsc.sort_key_val(keys, vv[k, pl.ds(g * 16, 16)])
                _, pi = plsc.sort_key_val(keys, vi[k, pl.ds(g * 16, 16)])
                permf.append(pf)
                permi.append(pi)
            for m in range(3):
                r = 16 * m + iota
                q = (r * 21846) >> 16
                kk = r - 3 * q
                outf = jnp.where(kk == 0, permf[0],
                                 jnp.where(kk == 1, permf[1], permf[2]))
                outi = jnp.where(kk == 0, permi[0],
                                 jnp.where(kk == 1, permi[1], permi[2]))
                o = 48 * g + 16 * m
                node = jbase + 16 * g + q
                ewv[pl.ds(o, 16)] = outf
                e0[0, pl.ds(o, 16)] = node
                e1[0, pl.ds(o, 16)] = outi
        o1 = pltpu.async_copy(ewv, ew_hbm.at[pl.ds(obase, epw)], s1)
        o2 = pltpu.async_copy(e0, ei_hbm.at[pl.ds(0, 1), pl.ds(obase, epw)], s2)
        o3 = pltpu.async_copy(e1, ei_hbm.at[pl.ds(1, 1), pl.ds(obase, epw)], s1)
        o1.wait()
        o2.wait()
        o3.wait()

    ew, ei = _sc(x, xi)
    return ei, ew


def kernel(features, mask, node_embeds, hyper_embeds):
    seq_len = min(features.shape[1], node_embeds.shape[0])
    ne = node_embeds[:seq_len]
    dim = ne.shape[1]
    hnum = hyper_embeds.shape[0]
    nblk = seq_len // _BLOCK

    vals, idxs = pl.pallas_call(
        _hyper_kernel,
        grid=(nblk,),
        in_specs=[
            pl.BlockSpec((mask.shape[0], _BLOCK), lambda i: (0, i)),
            pl.BlockSpec((_BLOCK, dim), lambda i: (i, 0)),
            pl.BlockSpec((hnum, dim), lambda i: (0, 0)),
        ],
        out_specs=[
            pl.BlockSpec((_TOPK, _BLOCK), lambda i: (0, i)),
            pl.BlockSpec((_TOPK, _BLOCK), lambda i: (0, i)),
        ],
        out_shape=[
            jax.ShapeDtypeStruct((_TOPK, seq_len), jnp.float32),
            jax.ShapeDtypeStruct((_TOPK, seq_len), jnp.int32),
        ],
    )(mask, ne, hyper_embeds)

    edge_index, edge_weight = _edge_assemble_sc(vals, idxs, seq_len)
    return (edge_index, edge_weight)


# speedup vs baseline: 2.8594x; 1.9130x over previous
import numpy as np

import jax
import jax.numpy as jnp
from jax.experimental import pallas as pl
from jax.experimental.pallas import tpu as pltpu

_ALPHA = 1.0
_TOPK = 3
_BLOCK = 1024
_NEG = -1e9
_L = 128


def _perm_mats():
    p = np.zeros((_TOPK, _TOPK * _L, _L), np.float32)
    for w in range(_TOPK):
        for c in range(_L):
            q = _L * w + c
            p[w, _L * (q % _TOPK) + q // _TOPK, c] = 1.0
    return jnp.asarray(p)


def _hyper_kernel(mask_ref, ne_ref, hy_ref, p_ref, ew_ref, ei_ref):
    i = pl.program_id(0)
    b = ne_ref.shape[0]
    rows_out = ew_ref.shape[0]
    ne = ne_ref[...]
    hy = hy_ref[...]
    simt = jax.lax.dot_general(
        hy, ne, (((1,), (1,)), ((), ())),
        preferred_element_type=jnp.float32)
    simt = jnp.maximum(_ALPHA * simt, 0.0)
    avg = jnp.mean(mask_ref[...], axis=0)
    simt = jnp.where(avg[None, :] < 0.5, _NEG, simt)
    m = jnp.max(simt, axis=0, keepdims=True)
    e = jnp.exp(simt - m)
    soft = e / jnp.sum(e, axis=0, keepdims=True)

    h = soft.shape[0]
    row = jax.lax.broadcasted_iota(jnp.int32, soft.shape, 0)
    v = soft
    vks, iks = [], []
    for k in range(_TOPK):
        mk = jnp.max(v, axis=0)
        ik = jnp.min(jnp.where(v == mk[None, :], row, h), axis=0)
        vks.append(mk)
        iks.append(ik)
        v = jnp.where(row == ik[None, :], -1.0, v)

    vf = jnp.concatenate([x.reshape(b // _L, _L) for x in vks], axis=1)
    vi = jnp.concatenate([x.astype(jnp.float32).reshape(b // _L, _L)
                          for x in iks], axis=1)
    for w in range(_TOPK):
        pw = p_ref[w]
        tf = jax.lax.dot_general(vf, pw, (((1,), (0,)), ((), ())),
                                 preferred_element_type=jnp.float32)
        ti = jax.lax.dot_general(vi, pw, (((1,), (0,)), ((), ())),
                                 preferred_element_type=jnp.float32)
        ew_ref[pl.Slice(w, b // _L, _TOPK), :] = tf
        ei_ref[pl.ds(1, 1), pl.Slice(w, b // _L, _TOPK), :] = (
            ti.astype(jnp.int32)[None])

    ploc = (jax.lax.broadcasted_iota(jnp.int32, (rows_out, _L), 0) * _L
            + jax.lax.broadcasted_iota(jnp.int32, (rows_out, _L), 1))
    ei_ref[pl.ds(0, 1), :, :] = (b * i + ((ploc * 21846) >> 16))[None]


def kernel(features, mask, node_embeds, hyper_embeds):
    seq_len = min(features.shape[1], node_embeds.shape[0])
    ne = node_embeds[:seq_len]
    dim = ne.shape[1]
    hnum = hyper_embeds.shape[0]
    nblk = seq_len // _BLOCK
    rpb = _TOPK * _BLOCK // _L
    nrows = nblk * rpb

    ew, ei = pl.pallas_call(
        _hyper_kernel,
        grid=(nblk,),
        in_specs=[
            pl.BlockSpec((mask.shape[0], _BLOCK), lambda i: (0, i)),
            pl.BlockSpec((_BLOCK, dim), lambda i: (i, 0)),
            pl.BlockSpec((hnum, dim), lambda i: (0, 0)),
            pl.BlockSpec((_TOPK, _TOPK * _L, _L), lambda i: (0, 0, 0)),
        ],
        out_specs=[
            pl.BlockSpec((rpb, _L), lambda i: (i, 0)),
            pl.BlockSpec((2, rpb, _L), lambda i: (0, i, 0)),
        ],
        out_shape=[
            jax.ShapeDtypeStruct((nrows, _L), jnp.float32),
            jax.ShapeDtypeStruct((2, nrows, _L), jnp.int32),
        ],
    )(mask, ne, hyper_embeds, _perm_mats())

    return (ei.reshape(2, -1), ew.reshape(-1))
